# baseline (speedup 1.0000x reference)
import functools

import jax
import jax.numpy as jnp
from jax.experimental import pallas as pl
from jax.experimental.pallas import tpu as pltpu

R = 4
N = 4096
DIN = 256
DOUT = 256
BM = 256


def _supports_body(x_ref, w_ref, b_ref, out_ref):
    s = jax.lax.dot_general(
        x_ref[...], w_ref[0], (((1,), (1,)), ((), ())),
        preferred_element_type=jnp.float32)
    out_ref[0] = s + b_ref[0]


def _agg_body(adj_ref, sup_ref, out_ref):
    r = pl.program_id(1)
    contrib = jnp.dot(adj_ref[0], sup_ref[r],
                      preferred_element_type=jnp.float32)

    @pl.when(r == 0)
    def _():
        out_ref[...] = contrib

    @pl.when(jnp.logical_and(r > 0, r < R - 1))
    def _():
        out_ref[...] = out_ref[...] + contrib

    @pl.when(r == R - 1)
    def _():
        out_ref[...] = jnp.tanh(out_ref[...] + contrib)


@jax.jit
def kernel(input, adjs, W, b):
    b3 = b.reshape(R, 1, DOUT)

    supports = pl.pallas_call(
        _supports_body,
        grid=(R,),
        in_specs=[
            pl.BlockSpec((N, DIN), lambda r: (0, 0)),
            pl.BlockSpec((1, DOUT, DIN), lambda r: (r, 0, 0)),
            pl.BlockSpec((1, 1, DOUT), lambda r: (r, 0, 0)),
        ],
        out_specs=pl.BlockSpec((1, N, DOUT), lambda r: (r, 0, 0)),
        out_shape=jax.ShapeDtypeStruct((R, N, DOUT), jnp.float32),
    )(input, W, b3)

    out = pl.pallas_call(
        _agg_body,
        grid=(N // BM, R),
        in_specs=[
            pl.BlockSpec((1, BM, N), lambda m, r: (r, m, 0)),
            pl.BlockSpec((R, N, DOUT), lambda m, r: (0, 0, 0)),
        ],
        out_specs=pl.BlockSpec((BM, DOUT), lambda m, r: (m, 0)),
        out_shape=jax.ShapeDtypeStruct((N, DOUT), jnp.float32),
        compiler_params=pltpu.CompilerParams(
            dimension_semantics=("parallel", "arbitrary"),
            vmem_limit_bytes=100 * 1024 * 1024,
        ),
    )(adjs, supports)
    return out


# speedup vs baseline: 1.0820x; 1.0820x over previous
import jax
import jax.numpy as jnp
from jax.experimental import pallas as pl
from jax.experimental.pallas import tpu as pltpu

R = 4
N = 4096
DIN = 256
DOUT = 256
BM = 256


def _rgcn_body(x_ref, w_ref, b_ref, adj_ref, out_ref, sup_ref):
    r = pl.program_id(0)
    m = pl.program_id(1)

    @pl.when(m == 0)
    def _():
        s = jax.lax.dot_general(
            x_ref[...], w_ref[r], (((1,), (1,)), ((), ())),
            preferred_element_type=jnp.float32)
        sup_ref[...] = s + b_ref[r]

    contrib = jnp.dot(adj_ref[0], sup_ref[...],
                      preferred_element_type=jnp.float32)
    rows = pl.ds(m * BM, BM)

    @pl.when(r == 0)
    def _():
        out_ref[rows, :] = contrib

    @pl.when(jnp.logical_and(r > 0, r < R - 1))
    def _():
        out_ref[rows, :] = out_ref[rows, :] + contrib

    @pl.when(r == R - 1)
    def _():
        out_ref[rows, :] = jnp.tanh(out_ref[rows, :] + contrib)


@jax.jit
def kernel(input, adjs, W, b):
    b3 = b.reshape(R, 1, DOUT)
    return pl.pallas_call(
        _rgcn_body,
        grid=(R, N // BM),
        in_specs=[
            pl.BlockSpec((N, DIN), lambda r, m: (0, 0)),
            pl.BlockSpec((R, DOUT, DIN), lambda r, m: (0, 0, 0)),
            pl.BlockSpec((R, 1, DOUT), lambda r, m: (0, 0, 0)),
            pl.BlockSpec((1, BM, N), lambda r, m: (r, m, 0)),
        ],
        out_specs=pl.BlockSpec((N, DOUT), lambda r, m: (0, 0)),
        out_shape=jax.ShapeDtypeStruct((N, DOUT), jnp.float32),
        scratch_shapes=[pltpu.VMEM((N, DOUT), jnp.float32)],
        compiler_params=pltpu.CompilerParams(
            dimension_semantics=("arbitrary", "arbitrary"),
            vmem_limit_bytes=100 * 1024 * 1024,
        ),
    )(input, W, b3, adjs)


# speedup vs baseline: 1.3369x; 1.2355x over previous
import jax
import jax.numpy as jnp
from jax.experimental import pallas as pl
from jax.experimental.pallas import tpu as pltpu

R = 4
N = 4096
DIN = 256
DOUT = 256
BM = 512


def _rgcn_body(x_ref, w_ref, b_ref, adj_ref, out_ref, sup_ref):
    r = pl.program_id(0)
    m = pl.program_id(1)

    @pl.when(m == 0)
    def _():
        s = jax.lax.dot_general(
            x_ref[...], w_ref[r], (((1,), (1,)), ((), ())),
            preferred_element_type=jnp.float32)
        sup_ref[...] = s + b_ref[r]

    contrib = jnp.dot(adj_ref[0], sup_ref[...],
                      preferred_element_type=jnp.float32)
    rows = pl.ds(m * BM, BM)

    @pl.when(r == 0)
    def _():
        out_ref[rows, :] = contrib

    @pl.when(jnp.logical_and(r > 0, r < R - 1))
    def _():
        out_ref[rows, :] = out_ref[rows, :] + contrib

    @pl.when(r == R - 1)
    def _():
        out_ref[rows, :] = jnp.tanh(out_ref[rows, :] + contrib)


@jax.jit
def kernel(input, adjs, W, b):
    b3 = b.reshape(R, 1, DOUT)
    return pl.pallas_call(
        _rgcn_body,
        grid=(R, N // BM),
        in_specs=[
            pl.BlockSpec((N, DIN), lambda r, m: (0, 0)),
            pl.BlockSpec((R, DOUT, DIN), lambda r, m: (0, 0, 0)),
            pl.BlockSpec((R, 1, DOUT), lambda r, m: (0, 0, 0)),
            pl.BlockSpec((1, BM, N), lambda r, m: (r, m, 0)),
        ],
        out_specs=pl.BlockSpec((N, DOUT), lambda r, m: (0, 0)),
        out_shape=jax.ShapeDtypeStruct((N, DOUT), jnp.float32),
        scratch_shapes=[pltpu.VMEM((N, DOUT), jnp.float32)],
        compiler_params=pltpu.CompilerParams(
            dimension_semantics=("arbitrary", "arbitrary"),
            vmem_limit_bytes=100 * 1024 * 1024,
        ),
    )(input, W, b3, adjs)
